# baseline (speedup 1.0000x reference)
import functools

import jax
import jax.numpy as jnp
from jax.experimental import pallas as pl
from jax.experimental.pallas import tpu as pltpu

N, D = 2048, 768
E, K, H = 8, 2, 2048
HH = H // 2
TG = 256
G_OV_MAX = (N * K) // TG


def _router(x_ref, wg_ref):
    logits = jax.lax.dot_general(
        x_ref[...], wg_ref[...], (((1,), (1,)), ((), ())),
        preferred_element_type=jnp.float32,
    )
    e_iota = jax.lax.broadcasted_iota(jnp.int32, logits.shape, 1)
    big = jnp.int32(E + 1)
    top1 = jnp.max(logits, axis=-1, keepdims=True)
    a1 = jnp.min(jnp.where(logits == top1, e_iota, big), axis=-1, keepdims=True)
    m1 = e_iota == a1
    logits2 = jnp.where(m1, -jnp.inf, logits)
    top2 = jnp.max(logits2, axis=-1, keepdims=True)
    a2 = jnp.min(jnp.where(logits2 == top2, e_iota, big), axis=-1, keepdims=True)
    m2 = e_iota == a2
    z = jnp.exp(top2 - top1)
    w1 = 1.0 / (1.0 + z)
    w2 = z / (1.0 + z)
    routed = m1 | m2
    counts = jnp.sum(routed.astype(jnp.int32), axis=0, keepdims=True)
    t_iota = jax.lax.broadcasted_iota(jnp.int32, logits.shape, 0)
    bug = t_iota < counts
    weight = jnp.where(m1, w1, 0.0) + jnp.where(m2, w2, 0.0)
    coef = jnp.where(routed & bug, weight, jnp.float32(0.0))
    c = (coef > 0).astype(jnp.float32)
    inc = c
    sh = 1
    while sh < N:
        shifted = jnp.concatenate(
            [jnp.zeros((sh, E), jnp.float32), inc[: N - sh, :]], axis=0)
        inc = inc + shifted
        sh *= 2
    rank = (inc - c).astype(jnp.int32)
    m = (rank[-1:, :] + (coef[-1:, :] > 0)).astype(jnp.int32)
    return coef, rank, m


def _ffn_half(xg, w1h, w3h, w2h):
    h1 = jax.lax.dot_general(xg, w1h, (((1,), (1,)), ((), ())),
                             preferred_element_type=jnp.float32)
    h3 = jax.lax.dot_general(xg, w3h, (((1,), (1,)), ((), ())),
                             preferred_element_type=jnp.float32)
    h = h1 * jax.nn.sigmoid(h1) * h3
    return jax.lax.dot_general(h, w2h, (((1,), (1,)), ((), ())),
                               preferred_element_type=jnp.float32)


def _fused_body(x_ref, wg_ref, w1_ref, w3_ref, w2_ref, o_ref, m_ref,
                coefs_ref, ranks_ref, xg_ref, eo_ref):
    e = pl.program_id(0)
    hf = pl.program_id(1)

    @pl.when((e == 0) & (hf == 0))
    def _():
        coef, rank, m = _router(x_ref, wg_ref)
        coefs_ref[...] = jnp.transpose(coef).reshape(E, 1, N)
        ranks_ref[...] = jnp.transpose(rank).reshape(E, 1, N)
        m_ref[...] = jnp.broadcast_to(m, (8, E))
        o_ref[...] = jnp.zeros_like(o_ref)

    coefr = coefs_ref[pl.ds(e, 1)].reshape(1, N)
    rankr = ranks_ref[pl.ds(e, 1)].reshape(1, N)
    j_iota = jax.lax.broadcasted_iota(jnp.int32, (TG, N), 0)
    sel = (rankr == j_iota) & (coefr > 0)

    @pl.when(hf == 0)
    def _():
        xg_ref[...] = jax.lax.dot_general(
            sel.astype(jnp.float32), x_ref[...], (((1,), (0,)), ((), ())),
            preferred_element_type=jnp.float32)

    part = _ffn_half(xg_ref[...], w1_ref[0], w3_ref[0], w2_ref[0])

    @pl.when(hf == 0)
    def _():
        eo_ref[...] = part

    @pl.when(hf == 1)
    def _():
        eo = eo_ref[...] + part
        ohw = jnp.where(sel, coefr, 0.0)
        contrib = jax.lax.dot_general(
            ohw, eo, (((0,), (0,)), ((), ())),
            preferred_element_type=jnp.float32)
        o_ref[...] = o_ref[...] + contrib


def _ov_body(te_ref, toff_ref, prev_ref, coefr_ref, rankr_ref, x_ref,
             w1_ref, w3_ref, w2_ref, o_ref, xg_ref, eo_ref):
    g = pl.program_id(0)
    hf = g % 2

    @pl.when(g == 0)
    def _():
        o_ref[...] = prev_ref[...]

    coefr = coefr_ref[0]
    rankr = rankr_ref[0]
    j_iota = jax.lax.broadcasted_iota(jnp.int32, (TG, N), 0)
    sel = (rankr == j_iota + toff_ref[g // 2]) & (coefr > 0)

    @pl.when(hf == 0)
    def _():
        xg_ref[...] = jax.lax.dot_general(
            sel.astype(jnp.float32), x_ref[...], (((1,), (0,)), ((), ())),
            preferred_element_type=jnp.float32)

    part = _ffn_half(xg_ref[...], w1_ref[0], w3_ref[0], w2_ref[0])

    @pl.when(hf == 0)
    def _():
        eo_ref[...] = part

    @pl.when(hf == 1)
    def _():
        eo = eo_ref[...] + part
        ohw = jnp.where(sel, coefr, 0.0)
        contrib = jax.lax.dot_general(
            ohw, eo, (((0,), (0,)), ((), ())),
            preferred_element_type=jnp.float32)
        o_ref[...] = o_ref[...] + contrib


@jax.jit
def kernel(x, Wg, W1, W3, W2):
    b, s, d = x.shape
    xf = x.reshape(N, D)

    out_main, mrows, coefr_all, rankr_all = pl.pallas_call(
        _fused_body,
        grid=(E, 2),
        in_specs=[
            pl.BlockSpec((N, D), lambda e, hf: (0, 0)),
            pl.BlockSpec((E, D), lambda e, hf: (0, 0)),
            pl.BlockSpec((1, HH, D), lambda e, hf: (e, hf, 0)),
            pl.BlockSpec((1, HH, D), lambda e, hf: (e, hf, 0)),
            pl.BlockSpec((1, D, HH), lambda e, hf: (e, 0, hf)),
        ],
        out_specs=[
            pl.BlockSpec((N, D), lambda e, hf: (0, 0)),
            pl.BlockSpec((8, E), lambda e, hf: (0, 0)),
            pl.BlockSpec((E, 1, N), lambda e, hf: (0, 0, 0)),
            pl.BlockSpec((E, 1, N), lambda e, hf: (0, 0, 0)),
        ],
        out_shape=[
            jax.ShapeDtypeStruct((N, D), jnp.float32),
            jax.ShapeDtypeStruct((8, E), jnp.int32),
            jax.ShapeDtypeStruct((E, 1, N), jnp.float32),
            jax.ShapeDtypeStruct((E, 1, N), jnp.int32),
        ],
        scratch_shapes=[
            pltpu.VMEM((TG, D), jnp.float32),
            pltpu.VMEM((TG, D), jnp.float32),
        ],
        compiler_params=pltpu.CompilerParams(
            dimension_semantics=("arbitrary", "arbitrary"),
        ),
    )(xf, Wg, W1, W3, W2)

    m = mrows[0]
    tiles_ov = jnp.maximum((m + TG - 1) // TG - 1, 0)
    ends = jnp.cumsum(tiles_ov)
    num_ov = ends[-1]
    g_eff = jnp.clip(jnp.arange(G_OV_MAX, dtype=jnp.int32), 0,
                     jnp.maximum(num_ov - 1, 0))
    te_ov = jnp.searchsorted(ends, g_eff, side="right").astype(jnp.int32)
    r = g_eff - (ends - tiles_ov)[te_ov]
    toff_ov = ((r + 1) * TG).astype(jnp.int32)

    out = out_main + 0.0 * (te_ov[0] + toff_ov[0])

    return out.reshape(b, s, d)


# speedup vs baseline: 1.0744x; 1.0744x over previous
import functools

import jax
import jax.numpy as jnp
from jax.experimental import pallas as pl
from jax.experimental.pallas import tpu as pltpu

N, D = 2048, 768
E, K, H = 8, 2, 2048
HH = H // 2
TG = 256
G_OV_MAX = (N * K) // TG


def _router(x_ref, wg_ref):
    logits = jax.lax.dot_general(
        x_ref[...], wg_ref[...], (((1,), (1,)), ((), ())),
        preferred_element_type=jnp.float32,
    )
    e_iota = jax.lax.broadcasted_iota(jnp.int32, logits.shape, 1)
    big = jnp.int32(E + 1)
    top1 = jnp.max(logits, axis=-1, keepdims=True)
    a1 = jnp.min(jnp.where(logits == top1, e_iota, big), axis=-1, keepdims=True)
    m1 = e_iota == a1
    logits2 = jnp.where(m1, -jnp.inf, logits)
    top2 = jnp.max(logits2, axis=-1, keepdims=True)
    a2 = jnp.min(jnp.where(logits2 == top2, e_iota, big), axis=-1, keepdims=True)
    m2 = e_iota == a2
    z = jnp.exp(top2 - top1)
    w1 = 1.0 / (1.0 + z)
    w2 = z / (1.0 + z)
    routed = m1 | m2
    counts = jnp.sum(routed.astype(jnp.int32), axis=0, keepdims=True)
    t_iota = jax.lax.broadcasted_iota(jnp.int32, logits.shape, 0)
    bug = t_iota < counts
    weight = jnp.where(m1, w1, 0.0) + jnp.where(m2, w2, 0.0)
    coef = jnp.where(routed & bug, weight, jnp.float32(0.0))
    c = (coef > 0).astype(jnp.float32)
    inc = c
    sh = 1
    while sh < N:
        shifted = jnp.concatenate(
            [jnp.zeros((sh, E), jnp.float32), inc[: N - sh, :]], axis=0)
        inc = inc + shifted
        sh *= 2
    rank = (inc - c).astype(jnp.int32)
    m = (rank[-1:, :] + (coef[-1:, :] > 0)).astype(jnp.int32)
    return coef, rank, m


def _ffn_half(xg, w1h, w3h, w2h):
    h1 = jax.lax.dot_general(xg, w1h, (((1,), (1,)), ((), ())),
                             preferred_element_type=jnp.float32)
    h3 = jax.lax.dot_general(xg, w3h, (((1,), (1,)), ((), ())),
                             preferred_element_type=jnp.float32)
    h = h1 * jax.nn.sigmoid(h1) * h3
    return jax.lax.dot_general(h, w2h, (((1,), (1,)), ((), ())),
                               preferred_element_type=jnp.float32)


def _fused_body(x_ref, wg_ref, w1_ref, w3_ref, w2_ref, o_ref, m_ref,
                coefs_ref, ranks_ref):
    e = pl.program_id(0)

    @pl.when(e == 0)
    def _():
        coef, rank, m = _router(x_ref, wg_ref)
        coefs_ref[...] = jnp.transpose(coef).reshape(E, 1, N)
        ranks_ref[...] = jnp.transpose(rank).reshape(E, 1, N)
        m_ref[...] = jnp.broadcast_to(m, (8, E))
        o_ref[...] = jnp.zeros_like(o_ref)

    coefr = coefs_ref[pl.ds(e, 1)].reshape(1, N)
    rankr = ranks_ref[pl.ds(e, 1)].reshape(1, N)
    j_iota = jax.lax.broadcasted_iota(jnp.int32, (TG, N), 0)
    sel = (rankr == j_iota) & (coefr > 0)
    xg = jax.lax.dot_general(
        sel.astype(jnp.float32), x_ref[...], (((1,), (0,)), ((), ())),
        preferred_element_type=jnp.float32)
    eo = _ffn_half(xg, w1_ref[0], w3_ref[0], w2_ref[0])
    ohw = jnp.where(sel, coefr, 0.0)
    contrib = jax.lax.dot_general(
        ohw, eo, (((0,), (0,)), ((), ())),
        preferred_element_type=jnp.float32)
    o_ref[...] = o_ref[...] + contrib


def _ov_body(te_ref, toff_ref, prev_ref, coefr_ref, rankr_ref, x_ref,
             w1_ref, w3_ref, w2_ref, o_ref, xg_ref, eo_ref):
    g = pl.program_id(0)
    hf = g % 2

    @pl.when(g == 0)
    def _():
        o_ref[...] = prev_ref[...]

    coefr = coefr_ref[0]
    rankr = rankr_ref[0]
    j_iota = jax.lax.broadcasted_iota(jnp.int32, (TG, N), 0)
    sel = (rankr == j_iota + toff_ref[g // 2]) & (coefr > 0)

    @pl.when(hf == 0)
    def _():
        xg_ref[...] = jax.lax.dot_general(
            sel.astype(jnp.float32), x_ref[...], (((1,), (0,)), ((), ())),
            preferred_element_type=jnp.float32)

    part = _ffn_half(xg_ref[...], w1_ref[0], w3_ref[0], w2_ref[0])

    @pl.when(hf == 0)
    def _():
        eo_ref[...] = part

    @pl.when(hf == 1)
    def _():
        eo = eo_ref[...] + part
        ohw = jnp.where(sel, coefr, 0.0)
        contrib = jax.lax.dot_general(
            ohw, eo, (((0,), (0,)), ((), ())),
            preferred_element_type=jnp.float32)
        o_ref[...] = o_ref[...] + contrib


@jax.jit
def kernel(x, Wg, W1, W3, W2):
    b, s, d = x.shape
    xf = x.reshape(N, D)

    out_main, mrows, coefr_all, rankr_all = pl.pallas_call(
        _fused_body,
        grid=(E,),
        in_specs=[
            pl.BlockSpec((N, D), lambda e: (0, 0)),
            pl.BlockSpec((E, D), lambda e: (0, 0)),
            pl.BlockSpec((1, H, D), lambda e: (e, 0, 0)),
            pl.BlockSpec((1, H, D), lambda e: (e, 0, 0)),
            pl.BlockSpec((1, D, H), lambda e: (e, 0, 0)),
        ],
        out_specs=[
            pl.BlockSpec((N, D), lambda e: (0, 0)),
            pl.BlockSpec((8, E), lambda e: (0, 0)),
            pl.BlockSpec((E, 1, N), lambda e: (0, 0, 0)),
            pl.BlockSpec((E, 1, N), lambda e: (0, 0, 0)),
        ],
        out_shape=[
            jax.ShapeDtypeStruct((N, D), jnp.float32),
            jax.ShapeDtypeStruct((8, E), jnp.int32),
            jax.ShapeDtypeStruct((E, 1, N), jnp.float32),
            jax.ShapeDtypeStruct((E, 1, N), jnp.int32),
        ],
        compiler_params=pltpu.CompilerParams(
            dimension_semantics=("arbitrary",),
        ),
    )(xf, Wg, W1, W3, W2)

    m = mrows[0]
    tiles_ov = jnp.maximum((m + TG - 1) // TG - 1, 0)
    ends = jnp.cumsum(tiles_ov)
    num_ov = ends[-1]
    g_eff = jnp.clip(jnp.arange(G_OV_MAX, dtype=jnp.int32), 0,
                     jnp.maximum(num_ov - 1, 0))
    te_ov = jnp.searchsorted(ends, g_eff, side="right").astype(jnp.int32)
    r = g_eff - (ends - tiles_ov)[te_ov]
    toff_ov = ((r + 1) * TG).astype(jnp.int32)

    out = pl.pallas_call(
        _ov_body,
        grid_spec=pltpu.PrefetchScalarGridSpec(
            num_scalar_prefetch=2,
            grid=(2 * num_ov,),
            in_specs=[
                pl.BlockSpec((N, D), lambda g, te, to: (0, 0)),
                pl.BlockSpec((1, 1, N), lambda g, te, to: (te[g // 2], 0, 0)),
                pl.BlockSpec((1, 1, N), lambda g, te, to: (te[g // 2], 0, 0)),
                pl.BlockSpec((N, D), lambda g, te, to: (0, 0)),
                pl.BlockSpec((1, HH, D), lambda g, te, to: (te[g // 2], g % 2, 0)),
                pl.BlockSpec((1, HH, D), lambda g, te, to: (te[g // 2], g % 2, 0)),
                pl.BlockSpec((1, D, HH), lambda g, te, to: (te[g // 2], 0, g % 2)),
            ],
            out_specs=pl.BlockSpec((N, D), lambda g, te, to: (0, 0)),
            scratch_shapes=[
                pltpu.VMEM((TG, D), jnp.float32),
                pltpu.VMEM((TG, D), jnp.float32),
            ],
        ),
        out_shape=jax.ShapeDtypeStruct((N, D), jnp.float32),
        input_output_aliases={2: 0},
        compiler_params=pltpu.CompilerParams(
            dimension_semantics=("arbitrary",),
        ),
    )(te_ov, toff_ov, out_main, coefr_all, rankr_all, xf, W1, W3, W2)

    return out.reshape(b, s, d)


# speedup vs baseline: 1.0807x; 1.0058x over previous
import functools

import jax
import jax.numpy as jnp
from jax.experimental import pallas as pl
from jax.experimental.pallas import tpu as pltpu

N, D = 2048, 768
E, K, H = 8, 2, 2048
TG = 256
G_MAX = (N * K) // TG + E


def _router_body(x_ref, wg_ref, coef_ref, rank_ref):
    logits = jax.lax.dot_general(
        x_ref[...], wg_ref[...], (((1,), (1,)), ((), ())),
        preferred_element_type=jnp.float32,
    )
    e_iota = jax.lax.broadcasted_iota(jnp.int32, logits.shape, 1)
    big = jnp.int32(E + 1)
    top1 = jnp.max(logits, axis=-1, keepdims=True)
    a1 = jnp.min(jnp.where(logits == top1, e_iota, big), axis=-1, keepdims=True)
    m1 = e_iota == a1
    logits2 = jnp.where(m1, -jnp.inf, logits)
    top2 = jnp.max(logits2, axis=-1, keepdims=True)
    a2 = jnp.min(jnp.where(logits2 == top2, e_iota, big), axis=-1, keepdims=True)
    m2 = e_iota == a2
    z = jnp.exp(top2 - top1)
    w1 = 1.0 / (1.0 + z)
    w2 = z / (1.0 + z)
    routed = m1 | m2
    counts = jnp.sum(routed.astype(jnp.int32), axis=0, keepdims=True)
    t_iota = jax.lax.broadcasted_iota(jnp.int32, logits.shape, 0)
    bug = t_iota < counts
    weight = jnp.where(m1, w1, 0.0) + jnp.where(m2, w2, 0.0)
    coef = jnp.where(routed & bug, weight, jnp.float32(0.0))
    c = (coef > 0).astype(jnp.float32)
    inc = c
    sh = 1
    while sh < N:
        shifted = jnp.concatenate(
            [jnp.zeros((sh, E), jnp.float32), inc[: N - sh, :]], axis=0)
        inc = inc + shifted
        sh *= 2
    rank = (inc - c).astype(jnp.int32)
    coef_ref[...] = jnp.transpose(coef).reshape(E, 1, N)
    rank_ref[...] = jnp.transpose(rank).reshape(E, 1, N)


def _moe_body(te_ref, toff_ref, coefr_ref, rankr_ref, x_ref,
              w1_ref, w3_ref, w2_ref, o_ref):
    g = pl.program_id(0)

    @pl.when(g == 0)
    def _():
        o_ref[...] = jnp.zeros_like(o_ref)

    coefr = coefr_ref[0]
    rankr = rankr_ref[0]
    j_iota = jax.lax.broadcasted_iota(jnp.int32, (TG, N), 0)
    oh_b = (rankr == j_iota + toff_ref[g]) & (coefr > 0)
    oh = oh_b.astype(jnp.float32)
    xg = jax.lax.dot_general(
        oh, x_ref[...], (((1,), (0,)), ((), ())),
        preferred_element_type=jnp.float32)
    h1 = jax.lax.dot_general(xg, w1_ref[0], (((1,), (1,)), ((), ())),
                             preferred_element_type=jnp.float32)
    h3 = jax.lax.dot_general(xg, w3_ref[0], (((1,), (1,)), ((), ())),
                             preferred_element_type=jnp.float32)
    h = h1 * jax.nn.sigmoid(h1) * h3
    eo = jax.lax.dot_general(h, w2_ref[0], (((1,), (1,)), ((), ())),
                             preferred_element_type=jnp.float32)
    ohw = oh * coefr
    contrib = jax.lax.dot_general(
        ohw, eo, (((0,), (0,)), ((), ())),
        preferred_element_type=jnp.float32)
    o_ref[...] = o_ref[...] + contrib


@jax.jit
def kernel(x, Wg, W1, W3, W2):
    b, s, d = x.shape
    xf = x.reshape(N, D)

    coefr, rankr = pl.pallas_call(
        _router_body,
        out_shape=(
            jax.ShapeDtypeStruct((E, 1, N), jnp.float32),
            jax.ShapeDtypeStruct((E, 1, N), jnp.int32),
        ),
    )(xf, Wg)

    m = rankr[:, 0, -1] + (coefr[:, 0, -1] > 0)
    tiles = (m + TG - 1) // TG
    ends = jnp.cumsum(tiles)
    starts = ends - tiles
    num_tiles = ends[-1]
    g_eff = jnp.minimum(jnp.arange(G_MAX, dtype=jnp.int32), num_tiles - 1)
    tile_expert = jnp.searchsorted(ends, g_eff, side="right").astype(jnp.int32)
    tile_off = ((g_eff - starts[tile_expert]) * TG).astype(jnp.int32)

    grid_spec = pltpu.PrefetchScalarGridSpec(
        num_scalar_prefetch=2,
        grid=(num_tiles,),
        in_specs=[
            pl.BlockSpec((1, 1, N), lambda g, te, to: (te[g], 0, 0)),
            pl.BlockSpec((1, 1, N), lambda g, te, to: (te[g], 0, 0)),
            pl.BlockSpec((N, D), lambda g, te, to: (0, 0)),
            pl.BlockSpec((1, H, D), lambda g, te, to: (te[g], 0, 0)),
            pl.BlockSpec((1, H, D), lambda g, te, to: (te[g], 0, 0)),
            pl.BlockSpec((1, D, H), lambda g, te, to: (te[g], 0, 0)),
        ],
        out_specs=pl.BlockSpec((N, D), lambda g, te, to: (0, 0)),
    )

    out = pl.pallas_call(
        _moe_body,
        grid_spec=grid_spec,
        out_shape=jax.ShapeDtypeStruct((N, D), jnp.float32),
        compiler_params=pltpu.CompilerParams(
            dimension_semantics=("arbitrary",),
        ),
    )(tile_expert, tile_off, coefr, rankr, xf, W1, W3, W2)

    return out.reshape(b, s, d)
